# Initial kernel scaffold; baseline (speedup 1.0000x reference)
import jax
import jax.numpy as jnp
from jax.experimental import pallas as pl


def kernel(predicted_positions, real_positions, real_expressions):
    raise NotImplementedError("write your pallas kernel here")


# speedup vs baseline: 1.2554x; 1.2554x over previous
import jax
import jax.numpy as jnp
from jax.experimental import pallas as pl
from jax.experimental.pallas import tpu as pltpu

_NB = 256


def _nn_kernel(pred_ref, realt_ref, expr_ref, out_ref):
    p = pred_ref[0]
    rt = realt_ref[0]
    q2 = jnp.sum(p * p, axis=1, keepdims=True)
    k2 = jnp.sum(rt * rt, axis=0, keepdims=True)
    qk = jax.lax.dot_general(
        p, rt, (((1,), (0,)), ((), ())),
        preferred_element_type=jnp.float32,
    )
    sq = q2 - 2.0 * qk + k2
    val = jnp.maximum(sq, 0.0)
    m = val.shape[1]
    minv = jnp.min(val, axis=1, keepdims=True)
    iota = jax.lax.broadcasted_iota(jnp.int32, val.shape, 1)
    idx = jnp.min(jnp.where(val == minv, iota, m), axis=1)
    onehot = (iota == idx[:, None]).astype(jnp.float32)
    out_ref[0] = jax.lax.dot_general(
        onehot, expr_ref[0], (((1,), (0,)), ((), ())),
        preferred_element_type=jnp.float32,
    )


def kernel(predicted_positions, real_positions, real_expressions):
    B, N, d = predicted_positions.shape
    M = real_positions.shape[1]
    G = real_expressions.shape[2]
    realt = jnp.transpose(real_positions, (0, 2, 1))
    grid = (B, N // _NB)
    return pl.pallas_call(
        _nn_kernel,
        grid=grid,
        in_specs=[
            pl.BlockSpec((1, _NB, d), lambda b, n: (b, n, 0)),
            pl.BlockSpec((1, d, M), lambda b, n: (b, 0, 0)),
            pl.BlockSpec((1, M, G), lambda b, n: (b, 0, 0)),
        ],
        out_specs=pl.BlockSpec((1, _NB, G), lambda b, n: (b, n, 0)),
        out_shape=jax.ShapeDtypeStruct((B, N, G), jnp.float32),
        compiler_params=pltpu.CompilerParams(
            dimension_semantics=("arbitrary", "arbitrary"),
        ),
    )(predicted_positions, realt, real_expressions)
